# Initial kernel scaffold; baseline (speedup 1.0000x reference)
import jax
import jax.numpy as jnp
from jax.experimental import pallas as pl


def kernel(x, table):
    raise NotImplementedError("write your pallas kernel here")


# speedup vs baseline: 3.3376x; 3.3376x over previous
import functools

import jax
import jax.numpy as jnp
from jax import lax
from jax.experimental import pallas as pl
from jax.experimental.pallas import tpu as pltpu
from jax.experimental.pallas import tpu_sc as plsc

VOCAB1 = 100001
D = 128
B = 4096 * 50
NC, NS = 2, 16
NW = NC * NS
C = 128
PER_W = B // NW
NCHUNK = PER_W // C


def _make_gather():
    mesh = plsc.VectorSubcoreMesh(core_axis_name="c", subcore_axis_name="s")

    @functools.partial(
        pl.kernel,
        mesh=mesh,
        out_type=jax.ShapeDtypeStruct((B, D), jnp.float32),
        scratch_types=[
            pltpu.VMEM((NCHUNK, C), jnp.int32),
            pltpu.VMEM((C, D), jnp.float32),
            pltpu.VMEM((C, D), jnp.float32),
            pltpu.SemaphoreType.DMA,
            pltpu.SemaphoreType.DMA,
        ],
    )
    def gather(x_hbm, table_hbm, out_hbm, idx_v, rows0, rows1, sem0, sem1):
        wid = lax.axis_index("s") * NC + lax.axis_index("c")
        base = wid * PER_W
        pltpu.sync_copy(x_hbm.at[wid], idx_v)

        def start(j, rows, sem):
            pltpu.async_copy(table_hbm.at[idx_v.at[j]], rows, sem)

        def finish(j, rows, sem):
            pltpu.make_async_copy(table_hbm.at[idx_v.at[j]], rows, sem).wait()
            pltpu.sync_copy(rows, out_hbm.at[pl.ds(base + j * C, C)])

        start(0, rows0, sem0)
        start(1, rows1, sem1)

        def body(i, carry):
            j = i * 2
            finish(j, rows0, sem0)
            start(j + 2, rows0, sem0)
            finish(j + 1, rows1, sem1)
            start(j + 3, rows1, sem1)
            return carry

        lax.fori_loop(0, (NCHUNK - 2) // 2, body, 0, unroll=False)
        finish(NCHUNK - 2, rows0, sem0)
        finish(NCHUNK - 1, rows1, sem1)

    return gather


_gather = _make_gather()


def kernel(x, table):
    xr = x.reshape(NW, NCHUNK, C)
    out = _gather(xr, table)
    return out.reshape(4096, 50, D)


# speedup vs baseline: 3.3516x; 1.0042x over previous
import functools

import jax
import jax.numpy as jnp
from jax import lax
from jax.experimental import pallas as pl
from jax.experimental.pallas import tpu as pltpu
from jax.experimental.pallas import tpu_sc as plsc

VOCAB1 = 100001
D = 128
B = 4096 * 50
NC, NS = 2, 16
NW = NC * NS
C = 128
PER_W = B // NW
NCHUNK = PER_W // C
NBUF = 5
GDEPTH = 3
GROUPS = NCHUNK // NBUF


def _make_gather():
    mesh = plsc.VectorSubcoreMesh(core_axis_name="c", subcore_axis_name="s")

    @functools.partial(
        pl.kernel,
        mesh=mesh,
        out_type=jax.ShapeDtypeStruct((B, D), jnp.float32),
        scratch_types=(
            [pltpu.VMEM((NCHUNK, C), jnp.int32)]
            + [pltpu.VMEM((C, D), jnp.float32) for _ in range(NBUF)]
            + [pltpu.SemaphoreType.DMA for _ in range(2 * NBUF)]
        ),
    )
    def gather(x_hbm, table_hbm, out_hbm, idx_v, *bufs_and_sems):
        rows = bufs_and_sems[:NBUF]
        sem_g = bufs_and_sems[NBUF:2 * NBUF]
        sem_o = bufs_and_sems[2 * NBUF:]
        wid = lax.axis_index("s") * NC + lax.axis_index("c")
        base = wid * PER_W
        pltpu.sync_copy(x_hbm.at[wid], idx_v)

        def g_start(j, b):
            pltpu.async_copy(table_hbm.at[idx_v.at[j]], rows[b], sem_g[b])

        def g_wait(j, b):
            pltpu.make_async_copy(
                table_hbm.at[idx_v.at[j]], rows[b], sem_g[b]).wait()

        def o_start(j, b):
            pltpu.async_copy(
                rows[b], out_hbm.at[pl.ds(base + j * C, C)], sem_o[b])

        def o_wait(j, b):
            pltpu.make_async_copy(
                rows[b], out_hbm.at[pl.ds(base + j * C, C)], sem_o[b]).wait()

        def step(j, b):
            if isinstance(j, int):
                if j - (NBUF - GDEPTH) >= 0:
                    o_wait(j - (NBUF - GDEPTH), (b + GDEPTH) % NBUF)
                if j + GDEPTH < NCHUNK:
                    g_start(j + GDEPTH, (b + GDEPTH) % NBUF)
            else:
                o_wait(j - (NBUF - GDEPTH), (b + GDEPTH) % NBUF)
                g_start(j + GDEPTH, (b + GDEPTH) % NBUF)
            g_wait(j, b)
            o_start(j, b)

        for b in range(GDEPTH):
            g_start(b, b)
        for b in range(NBUF):
            step(b, b)

        def body(g, carry):
            j0 = g * NBUF
            for b in range(NBUF):
                step(j0 + b, b)
            return carry

        lax.fori_loop(1, GROUPS - 1, body, 0, unroll=False)

        for b in range(NBUF):
            step((GROUPS - 1) * NBUF + b, b)
        for j in range(NCHUNK - (NBUF - GDEPTH), NCHUNK):
            o_wait(j, j % NBUF)

    return gather


_gather = _make_gather()


def kernel(x, table):
    xr = x.reshape(NW, NCHUNK, C)
    out = _gather(xr, table)
    return out.reshape(4096, 50, D)


# speedup vs baseline: 5.9509x; 1.7755x over previous
import functools

import jax
import jax.numpy as jnp
from jax import lax
from jax.experimental import pallas as pl
from jax.experimental.pallas import tpu as pltpu
from jax.experimental.pallas import tpu_sc as plsc

VOCAB1 = 100001
D = 128
S = 50
NB = 4096
NC, NS = 2, 16
NW = NC * NS
E = 2
C = E * S
B_PER_W = NB // NW
NCHUNK = B_PER_W // E
NBUF = 4
GDEPTH = 2
GROUPS = NCHUNK // NBUF


def _make_gather():
    mesh = plsc.VectorSubcoreMesh(core_axis_name="c", subcore_axis_name="s")

    @functools.partial(
        pl.kernel,
        mesh=mesh,
        out_type=jax.ShapeDtypeStruct((NB, S, D), jnp.float32),
        scratch_types=(
            [pltpu.VMEM((NCHUNK, C), jnp.int32)]
            + [pltpu.VMEM((C, D), jnp.float32) for _ in range(NBUF)]
            + [pltpu.SemaphoreType.DMA for _ in range(2 * NBUF)]
        ),
    )
    def gather(x_hbm, table_hbm, out_hbm, idx_v, *bufs_and_sems):
        rows = bufs_and_sems[:NBUF]
        sem_g = bufs_and_sems[NBUF:2 * NBUF]
        sem_o = bufs_and_sems[2 * NBUF:]
        wid = lax.axis_index("s") * NC + lax.axis_index("c")
        ebase = wid * B_PER_W
        pltpu.sync_copy(x_hbm.at[wid], idx_v)

        def g_start(j, b):
            pltpu.async_copy(table_hbm.at[idx_v.at[j]], rows[b], sem_g[b])

        def g_wait(j, b):
            pltpu.make_async_copy(
                table_hbm.at[idx_v.at[j]], rows[b], sem_g[b]).wait()

        def o_start(j, b):
            bb = ebase + j * E
            pltpu.async_copy(rows[b].at[pl.ds(0, S)], out_hbm.at[bb], sem_o[b])
            pltpu.async_copy(
                rows[b].at[pl.ds(S, S)], out_hbm.at[bb + 1], sem_o[b])

        def o_wait(j, b):
            bb = ebase + j * E
            pltpu.make_async_copy(
                rows[b].at[pl.ds(0, S)], out_hbm.at[bb], sem_o[b]).wait()
            pltpu.make_async_copy(
                rows[b].at[pl.ds(S, S)], out_hbm.at[bb + 1], sem_o[b]).wait()

        def step(j, b):
            if isinstance(j, int):
                if j - (NBUF - GDEPTH) >= 0:
                    o_wait(j - (NBUF - GDEPTH), (b + GDEPTH) % NBUF)
                if j + GDEPTH < NCHUNK:
                    g_start(j + GDEPTH, (b + GDEPTH) % NBUF)
            else:
                o_wait(j - (NBUF - GDEPTH), (b + GDEPTH) % NBUF)
                g_start(j + GDEPTH, (b + GDEPTH) % NBUF)
            g_wait(j, b)
            o_start(j, b)

        for b in range(GDEPTH):
            g_start(b, b)
        for b in range(NBUF):
            step(b, b)

        def body(g, carry):
            j0 = g * NBUF
            for b in range(NBUF):
                step(j0 + b, b)
            return carry

        lax.fori_loop(1, GROUPS - 1, body, 0, unroll=False)

        for b in range(NBUF):
            step((GROUPS - 1) * NBUF + b, b)
        for j in range(NCHUNK - (NBUF - GDEPTH), NCHUNK):
            o_wait(j, j % NBUF)

    return gather


_gather = _make_gather()


def kernel(x, table):
    xr = x.reshape(NW, NCHUNK, C)
    return _gather(xr, table)


# speedup vs baseline: 10.6975x; 1.7976x over previous
import functools

import jax
import jax.numpy as jnp
from jax import lax
from jax.experimental import pallas as pl
from jax.experimental.pallas import tpu as pltpu
from jax.experimental.pallas import tpu_sc as plsc

VOCAB1 = 100001
D = 128
S = 50
NB = 4096
NC, NS = 2, 16
NW = NC * NS
C = 128
NCHUNK = S
NBUF = 5
GDEPTH = 3
GROUPS = NCHUNK // NBUF


def _make_gather():
    mesh = plsc.VectorSubcoreMesh(core_axis_name="c", subcore_axis_name="s")

    @functools.partial(
        pl.kernel,
        mesh=mesh,
        out_type=jax.ShapeDtypeStruct((S, NB, D), jnp.float32),
        scratch_types=(
            [pltpu.VMEM((NCHUNK, C), jnp.int32)]
            + [pltpu.VMEM((C, D), jnp.float32) for _ in range(NBUF)]
            + [pltpu.SemaphoreType.DMA for _ in range(2 * NBUF)]
        ),
    )
    def gather(x_hbm, table_hbm, out_hbm, idx_v, *bufs_and_sems):
        rows = bufs_and_sems[:NBUF]
        sem_g = bufs_and_sems[NBUF:2 * NBUF]
        sem_o = bufs_and_sems[2 * NBUF:]
        wid = lax.axis_index("s") * NC + lax.axis_index("c")
        bbase = wid * C
        pltpu.sync_copy(x_hbm.at[wid], idx_v)

        def g_start(j, b):
            pltpu.async_copy(table_hbm.at[idx_v.at[j]], rows[b], sem_g[b])

        def g_wait(j, b):
            pltpu.make_async_copy(
                table_hbm.at[idx_v.at[j]], rows[b], sem_g[b]).wait()

        def o_start(j, b):
            pltpu.async_copy(
                rows[b], out_hbm.at[j, pl.ds(bbase, C)], sem_o[b])

        def o_wait(j, b):
            pltpu.make_async_copy(
                rows[b], out_hbm.at[j, pl.ds(bbase, C)], sem_o[b]).wait()

        def step(j, b):
            if isinstance(j, int):
                if j - (NBUF - GDEPTH) >= 0:
                    o_wait(j - (NBUF - GDEPTH), (b + GDEPTH) % NBUF)
                if j + GDEPTH < NCHUNK:
                    g_start(j + GDEPTH, (b + GDEPTH) % NBUF)
            else:
                o_wait(j - (NBUF - GDEPTH), (b + GDEPTH) % NBUF)
                g_start(j + GDEPTH, (b + GDEPTH) % NBUF)
            g_wait(j, b)
            o_start(j, b)

        for b in range(GDEPTH):
            g_start(b, b)
        for b in range(NBUF):
            step(b, b)

        def body(g, carry):
            j0 = g * NBUF
            for b in range(NBUF):
                step(j0 + b, b)
            return carry

        lax.fori_loop(1, GROUPS - 1, body, 0, unroll=False)

        for b in range(NBUF):
            step((GROUPS - 1) * NBUF + b, b)
        for j in range(NCHUNK - (NBUF - GDEPTH), NCHUNK):
            o_wait(j, j % NBUF)

    return gather


_gather = _make_gather()


def kernel(x, table):
    xt = x.reshape(NW, C, S).transpose(0, 2, 1)
    out_sw = _gather(xt, table)
    return out_sw.transpose(1, 0, 2)
